# baseline (speedup 1.0000x reference)
import jax
import jax.numpy as jnp
from jax.experimental import pallas as pl
from jax.experimental.pallas import tpu as pltpu

_N = 100000
_SEG = 512
_T = 1000.0
_B = 2048
_R = _B // 128
_RX = 3 * _B // 128
_G = 49
_NPAD = _G * _B


def _inv_expm1(z):
    u = jnp.exp(z)
    d = u - 1.0
    em1 = jnp.where(d == 0.0, z, d * z / jnp.log(u))
    return 1.0 / em1


def _to_lane(pk, rows):
    return jnp.concatenate([pk[r:r + 1, :] for r in range(rows)], axis=1)


def _body(t_ref, seg_ref, dxf_ref, zxf_ref, dh_ref, zh_ref, out_ref, acc_ref):
    i = pl.program_id(0)

    @pl.when(i == 0)
    def _init():
        acc_ref[...] = jnp.zeros_like(acc_ref)

    t = t_ref[0]
    t_int = jnp.round(t * _T)
    s_t = t_int * (1.0 / _T)
    s_s = (t_int - 1.0) * (1.0 / _T)
    z_t = 1e-4 + 10.0 * s_t * s_t
    z_s = 1e-4 + 10.0 * s_s * s_s
    snr_w = _inv_expm1(z_s) - _inv_expm1(z_t)
    g_pk = jnp.where(t_int == 0.0, 0.0, snr_w)
    g = _to_lane(g_pk, _R)
    ids = _to_lane(seg_ref[0], _R)

    dnums = (((1,), (1,)), ((), ()))

    ex = dxf_ref[0] - zxf_ref[0]
    ex = (ex * ex).reshape(_R, 3, 128)
    l_iota = jax.lax.broadcasted_iota(jnp.int32, (128, 128), 0)
    k_iota = jax.lax.broadcasted_iota(jnp.int32, (128, 128), 1)
    sx_pk = jnp.zeros((_R, 128), jnp.float32)
    for j in range(3):
        p_j = ((l_iota + 128 * j) // 3 == k_iota).astype(jnp.float32)
        sx_pk = sx_pk + jnp.dot(ex[:, j, :], p_j,
                                preferred_element_type=jnp.float32)
    sq_x = _to_lane(sx_pk, _R)

    dh = dh_ref[...] - zh_ref[...]
    ones_h = jnp.ones((1, 128), jnp.float32)
    sq = sq_x + jax.lax.dot_general(ones_h, dh * dh, dnums,
                                    preferred_element_type=jnp.float32)
    pn = jnp.where(g == 0.0, 0.0, g * sq)

    p2 = jnp.concatenate([pn, jnp.ones_like(pn)], axis=0)
    one_hot = (jnp.broadcast_to(ids, (_SEG, _B))
               == jax.lax.broadcasted_iota(jnp.int32, (_SEG, _B), 0)
               ).astype(jnp.float32)
    acc_ref[...] += jax.lax.dot_general(
        one_hot, p2, dnums, preferred_element_type=jnp.float32)

    @pl.when(i == _G - 1)
    def _fini():
        seg_sum = acc_ref[:, 0:1]
        seg_cnt = acc_ref[:, 1:2]
        loss = seg_sum / jnp.maximum(seg_cnt, 1.0)
        out_ref[...] = jnp.sum(loss, axis=0, keepdims=True) * (1.0 / _SEG)


@jax.jit
def kernel(t, dx_t, dh_t, z_x, z_h, x, h, segment_ids):
    del x, h
    t3 = jnp.pad(t, (0, _NPAD - _N)).reshape(_G, _R, 128)
    seg3 = jnp.pad(segment_ids.astype(jnp.int32), (0, _NPAD - _N),
                   constant_values=_SEG).reshape(_G, _R, 128)
    dxf = jnp.pad(dx_t.reshape(3 * _N), (0, 3 * (_NPAD - _N))
                  ).reshape(_G, _RX, 128)
    zxf = jnp.pad(z_x.reshape(3 * _N), (0, 3 * (_NPAD - _N))
                  ).reshape(_G, _RX, 128)
    out = pl.pallas_call(
        _body,
        grid=(_G,),
        in_specs=[
            pl.BlockSpec((1, _R, 128), lambda i: (i, 0, 0)),
            pl.BlockSpec((1, _R, 128), lambda i: (i, 0, 0)),
            pl.BlockSpec((1, _RX, 128), lambda i: (i, 0, 0)),
            pl.BlockSpec((1, _RX, 128), lambda i: (i, 0, 0)),
            pl.BlockSpec((_B, 128), lambda i: (i, 0)),
            pl.BlockSpec((_B, 128), lambda i: (i, 0)),
        ],
        out_specs=pl.BlockSpec((1, 1), lambda i: (0, 0)),
        out_shape=jax.ShapeDtypeStruct((1, 1), jnp.float32),
        scratch_shapes=[pltpu.VMEM((_SEG, 2), jnp.float32)],
    )(t3, seg3, dxf, zxf, dh_t, z_h)
    return out[0, 0]


# speedup vs baseline: 2.8461x; 2.8461x over previous
import jax
import jax.numpy as jnp
from jax.experimental import pallas as pl
from jax.experimental.pallas import tpu as pltpu

_N = 100000
_SEG = 512
_T = 1000.0
_B = 2048
_R = _B // 128
_G = 49
_NPAD = _G * _B


def _inv_expm1(z):
    u = jnp.exp(z)
    d = u - 1.0
    em1 = jnp.where(d == 0.0, z, d * z / jnp.log(u))
    return 1.0 / em1


def _to_lane(pk):
    return jnp.concatenate([pk[r:r + 1, :] for r in range(_R)], axis=1)


def _body(t_ref, seg_ref, dxt_ref, zxt_ref, dh_ref, zh_ref, out_ref, acc_ref):
    i = pl.program_id(0)

    @pl.when(i == 0)
    def _init():
        acc_ref[...] = jnp.zeros_like(acc_ref)

    t = t_ref[0]
    t_int = jnp.round(t * _T)
    s_t = t_int * (1.0 / _T)
    s_s = (t_int - 1.0) * (1.0 / _T)
    z_t = 1e-4 + 10.0 * s_t * s_t
    z_s = 1e-4 + 10.0 * s_s * s_s
    snr_w = _inv_expm1(z_s) - _inv_expm1(z_t)
    g_pk = jnp.where(t_int == 0.0, 0.0, snr_w)
    g = _to_lane(g_pk)
    ids = _to_lane(seg_ref[0])

    ex = dxt_ref[...] - zxt_ref[...]
    ex = ex * ex
    sq_x = ex[0:1, :] + ex[1:2, :] + ex[2:3, :]

    dh = dh_ref[...] - zh_ref[...]
    ones_h = jnp.ones((1, 128), jnp.float32)
    dnums = (((1,), (1,)), ((), ()))
    sq = sq_x + jax.lax.dot_general(ones_h, dh * dh, dnums,
                                    preferred_element_type=jnp.float32)
    pn = jnp.where(g == 0.0, 0.0, g * sq)

    p2 = jnp.concatenate([pn, jnp.ones_like(pn)], axis=0)
    one_hot = (jnp.broadcast_to(ids, (_SEG, _B))
               == jax.lax.broadcasted_iota(jnp.int32, (_SEG, _B), 0)
               ).astype(jnp.float32)
    acc_ref[...] += jax.lax.dot_general(
        one_hot, p2, dnums, preferred_element_type=jnp.float32)

    @pl.when(i == _G - 1)
    def _fini():
        seg_sum = acc_ref[:, 0:1]
        seg_cnt = acc_ref[:, 1:2]
        loss = seg_sum / jnp.maximum(seg_cnt, 1.0)
        out_ref[...] = jnp.sum(loss, axis=0, keepdims=True) * (1.0 / _SEG)


@jax.jit
def kernel(t, dx_t, dh_t, z_x, z_h, x, h, segment_ids):
    del x, h
    t3 = jnp.pad(t, (0, _NPAD - _N)).reshape(_G, _R, 128)
    seg3 = jnp.pad(segment_ids.astype(jnp.int32), (0, _NPAD - _N),
                   constant_values=_SEG).reshape(_G, _R, 128)
    out = pl.pallas_call(
        _body,
        grid=(_G,),
        in_specs=[
            pl.BlockSpec((1, _R, 128), lambda i: (i, 0, 0)),
            pl.BlockSpec((1, _R, 128), lambda i: (i, 0, 0)),
            pl.BlockSpec((3, _B), lambda i: (0, i)),
            pl.BlockSpec((3, _B), lambda i: (0, i)),
            pl.BlockSpec((_B, 128), lambda i: (i, 0)),
            pl.BlockSpec((_B, 128), lambda i: (i, 0)),
        ],
        out_specs=pl.BlockSpec((1, 1), lambda i: (0, 0)),
        out_shape=jax.ShapeDtypeStruct((1, 1), jnp.float32),
        scratch_shapes=[pltpu.VMEM((_SEG, 2), jnp.float32)],
    )(t3, seg3, dx_t.T, z_x.T, dh_t, z_h)
    return out[0, 0]


# speedup vs baseline: 3.5430x; 1.2449x over previous
import jax
import jax.numpy as jnp
from jax.experimental import pallas as pl
from jax.experimental.pallas import tpu as pltpu

_N = 100000
_SEG = 512
_T = 1000.0
_B = 4096
_R = _B // 128
_G = 25
_NPAD = _G * _B


def _inv_expm1(z):
    u = jnp.exp(z)
    d = u - 1.0
    em1 = jnp.where(d == 0.0, z, d * z / jnp.log(u))
    return 1.0 / em1


def _to_lane(pk):
    return jnp.concatenate([pk[r:r + 1, :] for r in range(_R)], axis=1)


def _body(t_ref, seg_ref, dxt_ref, zxt_ref, dh_ref, zh_ref, out_ref, acc_ref):
    i = pl.program_id(0)

    @pl.when(i == 0)
    def _init():
        acc_ref[...] = jnp.zeros_like(acc_ref)

    t = t_ref[0]
    t_int = jnp.round(t * _T)
    s_t = t_int * (1.0 / _T)
    s_s = (t_int - 1.0) * (1.0 / _T)
    z_t = 1e-4 + 10.0 * s_t * s_t
    z_s = 1e-4 + 10.0 * s_s * s_s
    snr_w = _inv_expm1(z_s) - _inv_expm1(z_t)
    g_pk = jnp.where(t_int == 0.0, 0.0, snr_w)
    g = _to_lane(g_pk)
    ids = _to_lane(seg_ref[0])

    ex = dxt_ref[...] - zxt_ref[...]
    ex = ex * ex
    sq_x = ex[0:1, :] + ex[1:2, :] + ex[2:3, :]

    dh = dh_ref[...] - zh_ref[...]
    ones_h = jnp.ones((1, 128), jnp.float32)
    dnums = (((1,), (1,)), ((), ()))
    sq = sq_x + jax.lax.dot_general(ones_h, dh * dh, dnums,
                                    preferred_element_type=jnp.float32)
    pn = jnp.where(g == 0.0, 0.0, g * sq)

    p2 = jnp.concatenate([pn, jnp.ones_like(pn)], axis=0)
    one_hot = (jnp.broadcast_to(ids, (_SEG, _B))
               == jax.lax.broadcasted_iota(jnp.int32, (_SEG, _B), 0)
               ).astype(jnp.float32)
    acc_ref[...] += jax.lax.dot_general(
        one_hot, p2, dnums, preferred_element_type=jnp.float32)

    @pl.when(i == _G - 1)
    def _fini():
        seg_sum = acc_ref[:, 0:1]
        seg_cnt = acc_ref[:, 1:2]
        loss = seg_sum / jnp.maximum(seg_cnt, 1.0)
        out_ref[...] = jnp.sum(loss, axis=0, keepdims=True) * (1.0 / _SEG)


@jax.jit
def kernel(t, dx_t, dh_t, z_x, z_h, x, h, segment_ids):
    del x, h
    t3 = jnp.pad(t, (0, _NPAD - _N)).reshape(_G, _R, 128)
    seg3 = jnp.pad(segment_ids.astype(jnp.int32), (0, _NPAD - _N),
                   constant_values=_SEG).reshape(_G, _R, 128)
    out = pl.pallas_call(
        _body,
        grid=(_G,),
        in_specs=[
            pl.BlockSpec((1, _R, 128), lambda i: (i, 0, 0)),
            pl.BlockSpec((1, _R, 128), lambda i: (i, 0, 0)),
            pl.BlockSpec((3, _B), lambda i: (0, i)),
            pl.BlockSpec((3, _B), lambda i: (0, i)),
            pl.BlockSpec((_B, 128), lambda i: (i, 0)),
            pl.BlockSpec((_B, 128), lambda i: (i, 0)),
        ],
        out_specs=pl.BlockSpec((1, 1), lambda i: (0, 0)),
        out_shape=jax.ShapeDtypeStruct((1, 1), jnp.float32),
        scratch_shapes=[pltpu.VMEM((_SEG, 2), jnp.float32)],
    )(t3, seg3, dx_t.T, z_x.T, dh_t, z_h)
    return out[0, 0]


# speedup vs baseline: 3.8819x; 1.0956x over previous
import jax
import jax.numpy as jnp
from jax.experimental import pallas as pl
from jax.experimental.pallas import tpu as pltpu

_N = 100000
_SEG = 512
_T = 1000.0
_B = 8192
_R = _B // 128
_G = 13
_NPAD = _G * _B


def _inv_expm1(z):
    u = jnp.exp(z)
    d = u - 1.0
    em1 = jnp.where(d == 0.0, z, d * z / jnp.log(u))
    return 1.0 / em1


def _to_lane(pk):
    return jnp.concatenate([pk[r:r + 1, :] for r in range(_R)], axis=1)


def _body(t_ref, seg_ref, dxt_ref, zxt_ref, dh_ref, zh_ref, out_ref, acc_ref):
    i = pl.program_id(0)

    @pl.when(i == 0)
    def _init():
        acc_ref[...] = jnp.zeros_like(acc_ref)

    t = t_ref[0]
    t_int = jnp.round(t * _T)
    s_t = t_int * (1.0 / _T)
    s_s = (t_int - 1.0) * (1.0 / _T)
    z_t = 1e-4 + 10.0 * s_t * s_t
    z_s = 1e-4 + 10.0 * s_s * s_s
    snr_w = _inv_expm1(z_s) - _inv_expm1(z_t)
    g_pk = jnp.where(t_int == 0.0, 0.0, snr_w)
    g = _to_lane(g_pk)
    ids = _to_lane(seg_ref[0])

    ex = dxt_ref[...] - zxt_ref[...]
    ex = ex * ex
    sq_x = ex[0:1, :] + ex[1:2, :] + ex[2:3, :]

    dh = dh_ref[...] - zh_ref[...]
    ones_h = jnp.ones((1, 128), jnp.float32)
    dnums = (((1,), (1,)), ((), ()))
    sq = sq_x + jax.lax.dot_general(ones_h, dh * dh, dnums,
                                    preferred_element_type=jnp.float32)
    pn = jnp.where(g == 0.0, 0.0, g * sq)

    p2 = jnp.concatenate([pn, jnp.ones_like(pn)], axis=0)
    one_hot = (jnp.broadcast_to(ids, (_SEG, _B))
               == jax.lax.broadcasted_iota(jnp.int32, (_SEG, _B), 0)
               ).astype(jnp.float32)
    acc_ref[...] += jax.lax.dot_general(
        one_hot, p2, dnums, preferred_element_type=jnp.float32)

    @pl.when(i == _G - 1)
    def _fini():
        seg_sum = acc_ref[:, 0:1]
        seg_cnt = acc_ref[:, 1:2]
        loss = seg_sum / jnp.maximum(seg_cnt, 1.0)
        out_ref[...] = jnp.sum(loss, axis=0, keepdims=True) * (1.0 / _SEG)


@jax.jit
def kernel(t, dx_t, dh_t, z_x, z_h, x, h, segment_ids):
    del x, h
    t3 = jnp.pad(t, (0, _NPAD - _N)).reshape(_G, _R, 128)
    seg3 = jnp.pad(segment_ids.astype(jnp.int32), (0, _NPAD - _N),
                   constant_values=_SEG).reshape(_G, _R, 128)
    out = pl.pallas_call(
        _body,
        grid=(_G,),
        in_specs=[
            pl.BlockSpec((1, _R, 128), lambda i: (i, 0, 0)),
            pl.BlockSpec((1, _R, 128), lambda i: (i, 0, 0)),
            pl.BlockSpec((3, _B), lambda i: (0, i)),
            pl.BlockSpec((3, _B), lambda i: (0, i)),
            pl.BlockSpec((_B, 128), lambda i: (i, 0)),
            pl.BlockSpec((_B, 128), lambda i: (i, 0)),
        ],
        out_specs=pl.BlockSpec((1, 1), lambda i: (0, 0)),
        out_shape=jax.ShapeDtypeStruct((1, 1), jnp.float32),
        scratch_shapes=[pltpu.VMEM((_SEG, 2), jnp.float32)],
    )(t3, seg3, dx_t.T, z_x.T, dh_t, z_h)
    return out[0, 0]
